# Initial kernel scaffold; baseline (speedup 1.0000x reference)
import jax
import jax.numpy as jnp
from jax.experimental import pallas as pl


def kernel(features, rbf_expansion, neighbor_list, W1, b1, W2, b2, nbr_filter):
    raise NotImplementedError("write your pallas kernel here")


# speedup vs baseline: 8.9895x; 8.9895x over previous
import functools

import jax
import jax.numpy as jnp
from jax import lax
from jax.experimental import pallas as pl
from jax.experimental.pallas import tpu as pltpu
from jax.experimental.pallas import tpu_sc as plsc

F = 128
N_NBR = 64


_NUM_WORKERS = 32
_GATHER_CHUNK = 128


def _sc_gather_body(nchunks, table_hbm, idx_hbm, out_hbm, idx_v, buf, sem):
    wid = lax.axis_index("s") * 2 + lax.axis_index("c")
    rows_per_w = nchunks * _GATHER_CHUNK
    base = wid * rows_per_w
    pltpu.sync_copy(idx_hbm.at[pl.ds(base, rows_per_w)], idx_v)

    def chunk(i, carry):
        off = i * _GATHER_CHUNK
        pltpu.async_copy(
            table_hbm.at[idx_v.at[pl.ds(off, _GATHER_CHUNK)]], buf, sem
        ).wait()
        pltpu.sync_copy(buf, out_hbm.at[pl.ds(base + off, _GATHER_CHUNK)])
        return carry

    lax.fori_loop(0, nchunks, chunk, 0)


def _sc_gather(table, idx):
    m = idx.shape[0]
    rows_per_w = m // _NUM_WORKERS
    nchunks = rows_per_w // _GATHER_CHUNK
    mesh = plsc.VectorSubcoreMesh(core_axis_name="c", subcore_axis_name="s")
    kern = functools.partial(
        pl.kernel,
        mesh=mesh,
        out_type=jax.ShapeDtypeStruct((m, F), jnp.float32),
        scratch_types=[
            pltpu.VMEM((rows_per_w,), jnp.int32),
            pltpu.VMEM((_GATHER_CHUNK, F), jnp.float32),
            pltpu.SemaphoreType.DMA,
        ],
    )(functools.partial(_sc_gather_body, nchunks))
    return kern(table, idx)


_ROWS_PER_BLOCK = 2048


def _tc_body(rbf_ref, gath_ref, w1_ref, b1_ref, w2_ref, b2_ref, nf_ref,
             out_ref, attn_ref):
    rows = rbf_ref.shape[0]
    atoms = rows // N_NBR
    x = rbf_ref[...]
    h = jnp.tanh(
        jnp.dot(x, w1_ref[...], preferred_element_type=jnp.float32)
        + b1_ref[...]
    )
    filt = (
        jnp.dot(h, w2_ref[...], preferred_element_type=jnp.float32)
        + b2_ref[...]
    )
    fg = filt * gath_ref[...]
    fg3 = fg.reshape(atoms, N_NBR, F)
    logits = jnp.sum(fg3 * nf_ref[...].reshape(1, 1, F), axis=2)
    m = jnp.max(logits, axis=1, keepdims=True)
    e = jnp.exp(logits - m)
    attn = e / jnp.sum(e, axis=1, keepdims=True)
    out_ref[...] = jnp.sum(fg3 * attn[:, :, None], axis=1)
    attn_ref[...] = attn


def _tc_forward(rbf2, gath, w1, b1, w2, b2, nf):
    rows = rbf2.shape[0]
    nblocks = rows // _ROWS_PER_BLOCK
    atoms_per_block = _ROWS_PER_BLOCK // N_NBR
    return pl.pallas_call(
        _tc_body,
        grid=(nblocks,),
        in_specs=[
            pl.BlockSpec((_ROWS_PER_BLOCK, F), lambda i: (i, 0)),
            pl.BlockSpec((_ROWS_PER_BLOCK, F), lambda i: (i, 0)),
            pl.BlockSpec((F, F), lambda i: (0, 0)),
            pl.BlockSpec((1, F), lambda i: (0, 0)),
            pl.BlockSpec((F, F), lambda i: (0, 0)),
            pl.BlockSpec((1, F), lambda i: (0, 0)),
            pl.BlockSpec((1, F), lambda i: (0, 0)),
        ],
        out_specs=[
            pl.BlockSpec((atoms_per_block, F), lambda i: (i, 0)),
            pl.BlockSpec((atoms_per_block, N_NBR), lambda i: (i, 0)),
        ],
        out_shape=[
            jax.ShapeDtypeStruct((rows // N_NBR, F), jnp.float32),
            jax.ShapeDtypeStruct((rows // N_NBR, N_NBR), jnp.float32),
        ],
    )(rbf2, gath, w1, b1, w2, b2, nf)


def kernel(features, rbf_expansion, neighbor_list, W1, b1, W2, b2, nbr_filter):
    B, A, Fd = features.shape
    Nn = neighbor_list.shape[2]
    table = features.reshape(B * A, Fd)
    idx = (
        neighbor_list + (jnp.arange(B, dtype=jnp.int32) * A)[:, None, None]
    ).reshape(B * A * Nn)
    gath = _sc_gather(table, idx)
    rbf2 = rbf_expansion.reshape(B * A * Nn, -1)
    out2, attn2 = _tc_forward(
        rbf2,
        gath,
        W1,
        b1.reshape(1, Fd),
        W2,
        b2.reshape(1, Fd),
        nbr_filter.reshape(1, Fd),
    )
    return out2.reshape(B, A, Fd), attn2.reshape(B, A, Nn)
